# Initial kernel scaffold; baseline (speedup 1.0000x reference)
import jax
import jax.numpy as jnp
from jax.experimental import pallas as pl


def kernel(node_feature, edge_index, edge_feature, W1n, b1n, W1e, b1e, W2n, b2n, W2e, b2e):
    raise NotImplementedError("write your pallas kernel here")


# speedup vs baseline: 2.8684x; 2.8684x over previous
import functools

import jax
import jax.numpy as jnp
from jax import lax
from jax.experimental import pallas as pl
from jax.experimental.pallas import tpu as pltpu
import jax.experimental.pallas.tpu_sc as plsc

_NC = 2
_NS = 16
_NW = _NC * _NS
_CH = 128
_INTERPRET = False


def _sc_ef_body(NCH, efr, dstr, z_f, f_out, facc, dst_v, ef_v):
    c = lax.axis_index("c")
    s = lax.axis_index("s")
    wid = c * _NS + s
    zrows = facc.shape[0] // _NS
    pltpu.sync_copy(z_f, facc.at[pl.ds(s * zrows, zrows)])
    plsc.subcore_barrier()

    def body(j, carry):
        pltpu.sync_copy(dstr.at[wid, j], dst_v)
        pltpu.sync_copy(efr.at[wid, j], ef_v)
        pltpu.sync_copy(ef_v, facc.at[dst_v], add=True)
        return carry

    lax.fori_loop(0, NCH, body, 0)
    plsc.subcore_barrier()
    pltpu.sync_copy(facc.at[pl.ds(s * zrows, zrows)],
                    f_out.at[c, pl.ds(s * zrows, zrows)])


def _sc_seg_body(NCH, x_hbm, srcr, dstr, z_d, a_out,
                 acc, src_v, dst_v, rows_v, sem):
    c = lax.axis_index("c")
    s = lax.axis_index("s")
    wid = c * _NS + s
    zrows = acc.shape[0] // _NS
    pltpu.sync_copy(z_d, acc.at[pl.ds(s * zrows, zrows)])
    plsc.subcore_barrier()

    def body(j, carry):
        pltpu.sync_copy(srcr.at[wid, j], src_v)
        pltpu.sync_copy(dstr.at[wid, j], dst_v)
        pltpu.async_copy(x_hbm.at[src_v], rows_v, sem).wait()
        pltpu.sync_copy(rows_v, acc.at[dst_v], add=True)
        return carry

    lax.fori_loop(0, NCH, body, 0)
    plsc.subcore_barrier()
    pltpu.sync_copy(acc.at[pl.ds(s * zrows, zrows)],
                    a_out.at[c, pl.ds(s * zrows, zrows)])


def _make_sc_ef(NP, NCH, DEA):
    mesh = plsc.VectorSubcoreMesh(core_axis_name="c", subcore_axis_name="s",
                                  num_cores=_NC, num_subcores=_NS)
    return pl.kernel(
        functools.partial(_sc_ef_body, NCH),
        out_type=jax.ShapeDtypeStruct((_NC, NP, DEA), jnp.float32),
        mesh=mesh,
        scratch_types=[
            pltpu.VMEM_SHARED((NP, DEA), jnp.float32),
            pltpu.VMEM((_CH,), jnp.int32),
            pltpu.VMEM((_CH, DEA), jnp.float32),
        ],
        compiler_params=pltpu.CompilerParams(use_tc_tiling_on_sc=False),
        interpret=_INTERPRET,
    )


def _make_sc_seg(NP, NCH, D):
    mesh = plsc.VectorSubcoreMesh(core_axis_name="c", subcore_axis_name="s",
                                  num_cores=_NC, num_subcores=_NS)
    return pl.kernel(
        functools.partial(_sc_seg_body, NCH),
        out_type=jax.ShapeDtypeStruct((_NC, NP, D), jnp.float32),
        mesh=mesh,
        scratch_types=[
            pltpu.VMEM_SHARED((NP, D), jnp.float32),
            pltpu.VMEM((_CH,), jnp.int32),
            pltpu.VMEM((_CH,), jnp.int32),
            pltpu.VMEM((_CH, D), jnp.float32),
            pltpu.SemaphoreType.DMA,
        ],
        interpret=_INTERPRET,
    )


def _tc1_body(a_ref, f_ref, w1n_ref, w1ea_ref, w2ea_ref, x1_ref, efw2_ref):
    a = a_ref[0] + a_ref[1]
    f = f_ref[0] + f_ref[1]
    x1 = (jnp.dot(a, w1n_ref[...], preferred_element_type=jnp.float32, precision=lax.Precision.HIGHEST)
          + jnp.dot(f, w1ea_ref[...], preferred_element_type=jnp.float32, precision=lax.Precision.HIGHEST))
    x1_ref[...] = jnp.maximum(x1, 0.0)
    efw2_ref[...] = jnp.dot(f, w2ea_ref[...], preferred_element_type=jnp.float32, precision=lax.Precision.HIGHEST)


def _make_tc1(NP, D, DEA, H):
    BM = NP // 8
    grid = (8,)
    return pl.pallas_call(
        _tc1_body,
        grid=grid,
        in_specs=[
            pl.BlockSpec((_NC, BM, D), lambda i: (0, i, 0)),
            pl.BlockSpec((_NC, BM, DEA), lambda i: (0, i, 0)),
            pl.BlockSpec((D, H), lambda i: (0, 0)),
            pl.BlockSpec((DEA, H), lambda i: (0, 0)),
            pl.BlockSpec((DEA, H), lambda i: (0, 0)),
        ],
        out_specs=[
            pl.BlockSpec((BM, H), lambda i: (i, 0)),
            pl.BlockSpec((BM, H), lambda i: (i, 0)),
        ],
        out_shape=[jax.ShapeDtypeStruct((NP, H), jnp.float32),
                   jax.ShapeDtypeStruct((NP, H), jnp.float32)],
        interpret=_INTERPRET,
    )


def _tc2_body(N, BM, b_ref, efw2_ref, w2n_ref, out_ref):
    i = pl.program_id(0)
    b = b_ref[0] + b_ref[1]
    out2 = (jnp.dot(b, w2n_ref[...], preferred_element_type=jnp.float32, precision=lax.Precision.HIGHEST)
            + efw2_ref[...])
    rmax = jnp.max(out2, axis=1)
    rmin = jnp.min(out2, axis=1)
    rows = lax.broadcasted_iota(jnp.int32, (BM,), 0) + i * BM
    m = ((rmax != rmin) & (rows < N)).astype(jnp.float32)
    part = jnp.sum(out2 * m[:, None], axis=0)

    @pl.when(i == 0)
    def _():
        out_ref[...] = jnp.zeros_like(out_ref)

    out_ref[...] += part[None, :]


def _make_tc2(N, NP, D, H):
    BM = NP // 8
    grid = (8,)
    return pl.pallas_call(
        functools.partial(_tc2_body, N, BM),
        grid=grid,
        in_specs=[
            pl.BlockSpec((_NC, BM, D), lambda i: (0, i, 0)),
            pl.BlockSpec((BM, H), lambda i: (i, 0)),
            pl.BlockSpec((D, H), lambda i: (0, 0)),
        ],
        out_specs=pl.BlockSpec((1, H), lambda i: (0, 0)),
        out_shape=jax.ShapeDtypeStruct((1, H), jnp.float32),
        interpret=_INTERPRET,
    )


def kernel(node_feature, edge_index, edge_feature, W1n, b1n, W1e, b1e,
           W2n, b2n, W2e, b2e):
    N, D = node_feature.shape
    E, DE = edge_feature.shape
    H = W1n.shape[1]
    DEA = 32

    src = edge_index[0].astype(jnp.int32)
    dst = edge_index[1].astype(jnp.int32)

    epw = _NW * _CH
    Ep = ((E + epw - 1) // epw) * epw
    pad = Ep - E
    src = jnp.concatenate([src, jnp.zeros((pad,), jnp.int32)])
    dst = jnp.concatenate([dst, jnp.full((pad,), N, jnp.int32)])
    NCH = Ep // epw
    srcr = src.reshape(_NW, NCH, _CH)
    dstr = dst.reshape(_NW, NCH, _CH)

    ef_aug = (jnp.zeros((Ep, DEA), jnp.float32)
              .at[:E, :DE].set(edge_feature)
              .at[:E, DE].set(1.0))
    efr = ef_aug.reshape(_NW, NCH, _CH, DEA)

    NP = ((N + 1 + 127) // 128) * 128
    zrows = NP // _NS
    z_d = jnp.zeros((zrows, D), jnp.float32)
    z_f = jnp.zeros((zrows, DEA), jnp.float32)

    W1ea = jnp.zeros((DEA, H), jnp.float32).at[:DE].set(W1e).at[DE].set(b1n + b1e)
    W2ea = jnp.zeros((DEA, H), jnp.float32).at[:DE].set(W2e).at[DE].set(b2n + b2e)

    F = _make_sc_ef(NP, NCH, DEA)(efr, dstr, z_f)
    A = _make_sc_seg(NP, NCH, D)(node_feature, srcr, dstr, z_d)
    X1, EFW2 = _make_tc1(NP, D, DEA, H)(A, F, W1n, W1ea, W2ea)
    B = _make_sc_seg(NP, NCH, H)(X1, srcr, dstr, z_d)
    pooled = _make_tc2(N, NP, H, H)(B, EFW2, W2n)
    return pooled.reshape(H)
